# Initial kernel scaffold; baseline (speedup 1.0000x reference)
import jax
import jax.numpy as jnp
from jax.experimental import pallas as pl


def kernel(edges, nodes, receivers, senders):
    raise NotImplementedError("write your pallas kernel here")


# speedup vs baseline: 6.0825x; 6.0825x over previous
import functools
import jax
import jax.numpy as jnp
from jax import lax
from jax.experimental import pallas as pl
from jax.experimental.pallas import tpu as pltpu
from jax.experimental.pallas import tpu_sc as plsc

N_NODES = 100000
N_EDGES = 3200000
D = 16
BATCH = 128
ROWS = N_EDGES // BATCH
CHUNK_ROWS = 8
N_CHUNKS = ROWS // CHUNK_ROWS
NC, NS = 2, 16
NW = NC * NS
TRIPS = (N_CHUNKS + NW - 1) // NW
ACC_STRIPE = 6248
ACC_REM = N_NODES - NS * ACC_STRIPE


@functools.partial(
    pl.kernel,
    out_type=jax.ShapeDtypeStruct((NC, N_NODES, D), jnp.float32),
    mesh=plsc.VectorSubcoreMesh(core_axis_name="c", subcore_axis_name="s"),
    compiler_params=pltpu.CompilerParams(use_tc_tiling_on_sc=False),
    scratch_types=[
        pltpu.VMEM_SHARED((N_NODES, D), jnp.float32),
        pltpu.VMEM((CHUNK_ROWS, BATCH), jnp.int32),
        pltpu.VMEM((CHUNK_ROWS, BATCH, D), jnp.float32),
    ],
)
def _sc_scatter_add(edges_hbm, recv_hbm, zeros_hbm, out_hbm, acc, idx_v, edge_v):
    c = lax.axis_index("c")
    s = lax.axis_index("s")
    w = s * NC + c

    pltpu.sync_copy(
        zeros_hbm.at[pl.ds(0, ACC_STRIPE)],
        acc.at[pl.ds(s * ACC_STRIPE, ACC_STRIPE)],
    )

    @pl.when(s == NS - 1)
    def _zero_tail():
        pltpu.sync_copy(
            zeros_hbm.at[pl.ds(0, ACC_REM)],
            acc.at[pl.ds(NS * ACC_STRIPE, ACC_REM)],
        )

    plsc.subcore_barrier()

    def chunk_body(t, carry):
        cid = w + NW * t

        @pl.when(cid < N_CHUNKS)
        def _():
            r0 = cid * CHUNK_ROWS
            pltpu.sync_copy(recv_hbm.at[pl.ds(r0, CHUNK_ROWS)], idx_v)
            pltpu.sync_copy(edges_hbm.at[pl.ds(r0, CHUNK_ROWS)], edge_v)
            for j in range(CHUNK_ROWS):
                pltpu.sync_copy(edge_v.at[j], acc.at[idx_v.at[j]], add=True)

        return carry

    lax.fori_loop(0, TRIPS, chunk_body, 0)
    plsc.subcore_barrier()

    pltpu.sync_copy(
        acc.at[pl.ds(s * ACC_STRIPE, ACC_STRIPE)],
        out_hbm.at[c, pl.ds(s * ACC_STRIPE, ACC_STRIPE)],
    )

    @pl.when(s == NS - 1)
    def _write_tail():
        pltpu.sync_copy(
            acc.at[pl.ds(NS * ACC_STRIPE, ACC_REM)],
            out_hbm.at[c, pl.ds(NS * ACC_STRIPE, ACC_REM)],
        )


def _combine_body(p_ref, o_ref):
    o_ref[...] = p_ref[0] + p_ref[1]


def kernel(edges, nodes, receivers, senders):
    edges3 = edges.reshape(ROWS, BATCH, D)
    recv2 = receivers.reshape(ROWS, BATCH)
    zeros = jnp.zeros((ACC_STRIPE, D), jnp.float32)
    partials = _sc_scatter_add(edges3, recv2, zeros)

    flat = partials.reshape(NC, (N_NODES * D) // 128, 128)
    n_rows = flat.shape[1]
    out = pl.pallas_call(
        _combine_body,
        out_shape=jax.ShapeDtypeStruct((n_rows, 128), jnp.float32),
    )(flat)
    return out.reshape(N_NODES, D)


# speedup vs baseline: 6.8670x; 1.1290x over previous
import functools
import jax
import jax.numpy as jnp
from jax import lax
from jax.experimental import pallas as pl
from jax.experimental.pallas import tpu as pltpu
from jax.experimental.pallas import tpu_sc as plsc

N_NODES = 100000
N_EDGES = 3200000
D = 16
BATCH = 128
ROWS = N_EDGES // BATCH
CHUNK_ROWS = 4
N_CHUNKS = ROWS // CHUNK_ROWS
NC, NS = 2, 16
NW = NC * NS
NBUF = 3
TRIPS_MAX = (N_CHUNKS + NW - 1) // NW
OUTER = (TRIPS_MAX + NBUF - 1 + NBUF - 1) // NBUF
ACC_STRIPE = 6248
ACC_REM = N_NODES - NS * ACC_STRIPE


@functools.partial(
    pl.kernel,
    out_type=jax.ShapeDtypeStruct((NC, N_NODES, D), jnp.float32),
    mesh=plsc.VectorSubcoreMesh(core_axis_name="c", subcore_axis_name="s"),
    compiler_params=pltpu.CompilerParams(use_tc_tiling_on_sc=False),
    scratch_types=[
        pltpu.VMEM_SHARED((N_NODES, D), jnp.float32),
        pltpu.VMEM((NBUF, CHUNK_ROWS, BATCH), jnp.int32),
        pltpu.VMEM((NBUF, CHUNK_ROWS, BATCH, D), jnp.float32),
        pltpu.SemaphoreType.DMA((NBUF,)),
        pltpu.SemaphoreType.DMA((NBUF,)),
    ],
)
def _sc_scatter_add(edges_hbm, recv_hbm, zeros_hbm, out_hbm,
                    acc, idx_v, edge_v, lsem, ssem):
    c = lax.axis_index("c")
    s = lax.axis_index("s")
    w = s * NC + c
    n_trips = (N_CHUNKS - w + NW - 1) // NW

    def start_loads(i, b):
        r0 = (w + NW * i) * CHUNK_ROWS
        pltpu.async_copy(recv_hbm.at[pl.ds(r0, CHUNK_ROWS)], idx_v.at[b],
                         lsem.at[b])
        pltpu.async_copy(edges_hbm.at[pl.ds(r0, CHUNK_ROWS)], edge_v.at[b],
                         lsem.at[b])

    def wait_loads(i, b):
        r0 = (w + NW * i) * CHUNK_ROWS
        pltpu.make_async_copy(recv_hbm.at[pl.ds(r0, CHUNK_ROWS)], idx_v.at[b],
                              lsem.at[b]).wait()
        pltpu.make_async_copy(edges_hbm.at[pl.ds(r0, CHUNK_ROWS)], edge_v.at[b],
                              lsem.at[b]).wait()

    def fire_scatters(b):
        for j in range(CHUNK_ROWS):
            pltpu.async_copy(edge_v.at[b, j], acc.at[idx_v.at[b, j]],
                             ssem.at[b], add=True)

    def drain_scatters(b):
        for j in range(CHUNK_ROWS):
            pltpu.make_async_copy(edge_v.at[b, j], acc.at[idx_v.at[b, j]],
                                  ssem.at[b]).wait()

    start_loads(0, 0)
    pltpu.sync_copy(
        zeros_hbm.at[pl.ds(0, ACC_STRIPE)],
        acc.at[pl.ds(s * ACC_STRIPE, ACC_STRIPE)],
    )

    @pl.when(s == NS - 1)
    def _zero_tail():
        pltpu.sync_copy(
            zeros_hbm.at[pl.ds(0, ACC_REM)],
            acc.at[pl.ds(NS * ACC_STRIPE, ACC_REM)],
        )

    plsc.subcore_barrier()

    def outer(t, carry):
        for b in range(NBUF):
            i = t * NBUF + b

            @pl.when((i >= NBUF - 1) & (i - (NBUF - 1) < n_trips))
            def _drain():
                drain_scatters((b + 1) % NBUF)

            @pl.when(i + 1 < n_trips)
            def _prefetch():
                start_loads(i + 1, (b + 1) % NBUF)

            @pl.when(i < n_trips)
            def _process():
                wait_loads(i, b)
                fire_scatters(b)

        return carry

    lax.fori_loop(0, OUTER, outer, 0)
    plsc.subcore_barrier()

    pltpu.sync_copy(
        acc.at[pl.ds(s * ACC_STRIPE, ACC_STRIPE)],
        out_hbm.at[c, pl.ds(s * ACC_STRIPE, ACC_STRIPE)],
    )

    @pl.when(s == NS - 1)
    def _write_tail():
        pltpu.sync_copy(
            acc.at[pl.ds(NS * ACC_STRIPE, ACC_REM)],
            out_hbm.at[c, pl.ds(NS * ACC_STRIPE, ACC_REM)],
        )


def _combine_body(p_ref, o_ref):
    o_ref[...] = p_ref[0] + p_ref[1]


def kernel(edges, nodes, receivers, senders):
    edges3 = edges.reshape(ROWS, BATCH, D)
    recv2 = receivers.reshape(ROWS, BATCH)
    zeros = jnp.zeros((ACC_STRIPE, D), jnp.float32)
    partials = _sc_scatter_add(edges3, recv2, zeros)

    flat = partials.reshape(NC, (N_NODES * D) // 128, 128)
    n_rows = flat.shape[1]
    out = pl.pallas_call(
        _combine_body,
        out_shape=jax.ShapeDtypeStruct((n_rows, 128), jnp.float32),
    )(flat)
    return out.reshape(N_NODES, D)
